# baseline (speedup 1.0000x reference)
import functools

import jax
import jax.numpy as jnp
from jax import lax
from jax.experimental import pallas as pl
from jax.experimental.pallas import tpu as pltpu
from jax.experimental.pallas import tpu_sc as plsc

VOCAB = 1000000
EMB_DIM = 32
BATCH = 4096
HIST = 200

NUM_CORES = 2
NUM_SUBCORES = 16
LANES = 16

BTILE = 128
HBLOCK = 8
NUM_HBLOCKS = HIST // HBLOCK

VTILE = 128
FULL_VCOLS = VOCAB // VTILE
VREM = VOCAB - FULL_VCOLS * VTILE
COLS_PER_SUB = (FULL_VCOLS + NUM_SUBCORES - 1) // NUM_SUBCORES
LINES = VOCAB * EMB_DIM // VTILE
LINES_PER_VCOL = VTILE * EMB_DIM // VTILE


def _emb_kernel(w_t, x_t, w_tail, out, wl, tbuf, trb, idx_t, line_i, quar,
                gbuf, pbuf, sem_t0, sem_t1, sem_w0, sem_w1,
                sem_g0, sem_g1, sem_o):
    c = lax.axis_index("c")
    s = lax.axis_index("s")
    wid = s * NUM_CORES + c
    b0 = wid * BTILE

    iota = lax.iota(jnp.int32, LANES)
    lane32 = iota * EMB_DIM
    sem_t = (sem_t0, sem_t1)
    sem_w = (sem_w0, sem_w1)
    sem_g = (sem_g0, sem_g1)

    def p1_tc(kk):
        return lax.rem(s + kk * NUM_SUBCORES + c * (FULL_VCOLS // 2),
                       FULL_VCOLS)

    def p1_stage(kk, parity):
        return pltpu.make_async_copy(
            w_t.at[:, pl.ds(p1_tc(kk) * VTILE, VTILE)],
            tbuf.at[parity], sem_t[parity])

    def p1_wout(kk, parity):
        return pltpu.make_async_copy(
            trb.at[parity],
            wl.at[pl.ds(p1_tc(kk) * LINES_PER_VCOL, LINES_PER_VCOL)],
            sem_w[parity])

    p1_stage(0, 0).start()

    def vcol_body(kk, carry):
        for parity in range(2):

            @pl.when(lax.rem(kk, 2) == parity)
            def _():
                p1_stage(kk, parity).wait()

                @pl.when(kk + 1 < COLS_PER_SUB)
                def _():
                    p1_stage(kk + 1, 1 - parity).start()

                @pl.when(kk >= 2)
                def _():
                    p1_wout(kk - 2, parity).wait()

                t = tbuf.at[parity]
                for d in range(EMB_DIM):
                    evec = lax.bitwise_and(iota + d, EMB_DIM - 1)
                    base = lane32 + evec
                    for jb in range(VTILE // LANES):
                        o = base + jb * (LANES * EMB_DIM)
                        rowv = lax.shift_right_logical(o, 7)
                        colv = lax.bitwise_and(o, VTILE - 1)
                        vals = plsc.load_gather(t, [evec, jb * LANES + iota])
                        plsc.store_scatter(trb.at[parity], [rowv, colv], vals)
                p1_wout(kk, parity).start()

        return carry

    lax.fori_loop(0, COLS_PER_SUB, vcol_body, 0)
    p1_wout(COLS_PER_SUB - 2, (COLS_PER_SUB - 2) % 2).wait()
    p1_wout(COLS_PER_SUB - 1, (COLS_PER_SUB - 1) % 2).wait()

    @pl.when(s == 0)
    def _tail():
        pltpu.sync_copy(w_tail, tbuf.at[0, pl.ds(0, 16), :])
        pltpu.sync_copy(tbuf.at[0, pl.ds(0, 16), :],
                        wl.at[pl.ds(FULL_VCOLS * LINES_PER_VCOL, 16)])

    plsc.subcore_barrier()

    def hblock_body(hb, carry):
        pltpu.sync_copy(
            x_t.at[pl.ds(hb * HBLOCK, HBLOCK), pl.ds(b0, BTILE)], idx_t)

        for r in range(HBLOCK):
            for j in range(BTILE // LANES):
                v = idx_t[r, pl.ds(j * LANES, LANES)]
                line_i[r, pl.ds(j * LANES, LANES)] = (
                    lax.shift_right_logical(v, 2))
                quar[r, pl.ds(j * LANES, LANES)] = lax.bitwise_and(v, 3)

        pending = pltpu.async_copy(wl.at[line_i.at[0]], gbuf.at[0], sem_g[0])
        for r in range(HBLOCK):
            pending.wait()
            if r + 1 < HBLOCK:
                pending = pltpu.async_copy(
                    wl.at[line_i.at[r + 1]], gbuf.at[(r + 1) % 2],
                    sem_g[(r + 1) % 2])
            g = gbuf.at[r % 2]

            pr = pbuf.at[r]

            def jblk_body(jb, carry2):
                j0 = jb * LANES
                jvec = j0 + iota
                colv = quar[r, pl.ds(j0, LANES)] * EMB_DIM
                for d in range(EMB_DIM):
                    evec = lax.bitwise_and(iota + d, EMB_DIM - 1)
                    vals = plsc.load_gather(g, [jvec, colv + evec])
                    plsc.store_scatter(pr, [evec, jvec], vals)
                return carry2

            lax.fori_loop(0, BTILE // LANES, jblk_body, 0)
        outs = []
        for r in range(HBLOCK):
            outs.append(pltpu.async_copy(
                pbuf.at[r],
                out.at[hb * HBLOCK + r, slice(None), pl.ds(b0, BTILE)],
                sem_o))
        for cp in outs:
            cp.wait()
        return carry

    lax.fori_loop(0, NUM_HBLOCKS, hblock_body, 0)


def kernel(x, weight):
    w_t = weight.T
    x_t = x.T
    w_tail = weight[FULL_VCOLS * VTILE:, :].reshape(16, VTILE)
    mesh = plsc.VectorSubcoreMesh(core_axis_name="c", subcore_axis_name="s")
    out3d, _ = pl.kernel(
        _emb_kernel,
        mesh=mesh,
        out_type=(
            jax.ShapeDtypeStruct((HIST, EMB_DIM, BATCH), jnp.float32),
            jax.ShapeDtypeStruct((LINES, VTILE), jnp.float32),
        ),
        scratch_types=[
            pltpu.VMEM((2, EMB_DIM, VTILE), jnp.float32),
            pltpu.VMEM((2, LINES_PER_VCOL, VTILE), jnp.float32),
            pltpu.VMEM((HBLOCK, BTILE), jnp.int32),
            pltpu.VMEM((HBLOCK, BTILE), jnp.int32),
            pltpu.VMEM((HBLOCK, BTILE), jnp.int32),
            pltpu.VMEM((2, BTILE, VTILE), jnp.float32),
            pltpu.VMEM((HBLOCK, EMB_DIM, BTILE), jnp.float32),
            pltpu.SemaphoreType.DMA,
            pltpu.SemaphoreType.DMA,
            pltpu.SemaphoreType.DMA,
            pltpu.SemaphoreType.DMA,
            pltpu.SemaphoreType.DMA,
            pltpu.SemaphoreType.DMA,
            pltpu.SemaphoreType.DMA,
        ],
        compiler_params=pltpu.CompilerParams(needs_layout_passes=False),
    )(w_t, x_t, w_tail)
    return jnp.transpose(out3d, (2, 0, 1))


# speedup vs baseline: 1.1877x; 1.1877x over previous
import functools

import jax
import jax.numpy as jnp
from jax import lax
from jax.experimental import pallas as pl
from jax.experimental.pallas import tpu as pltpu
from jax.experimental.pallas import tpu_sc as plsc

VOCAB = 1000000
EMB_DIM = 32
BATCH = 4096
HIST = 200

NUM_CORES = 2
NUM_SUBCORES = 16
NUM_WORKERS = NUM_CORES * NUM_SUBCORES

ROWS_TOTAL = BATCH * HIST
ROWS_PER_WORKER = ROWS_TOTAL // NUM_WORKERS

IDX_MINOR = 128
GATHERS_PER_CHUNK = 8
CHUNK_ROWS = IDX_MINOR * GATHERS_PER_CHUNK
NUM_CHUNKS = ROWS_PER_WORKER // CHUNK_ROWS
IDX_ROWS_PER_WORKER = ROWS_PER_WORKER // IDX_MINOR


def _gather_kernel(weight_hbm, idx_hbm, out_hbm, idx_v, rows_v, sem_g, sem_o):
    c = lax.axis_index("c")
    s = lax.axis_index("s")
    wid = s * NUM_CORES + c

    pltpu.sync_copy(idx_hbm.at[pl.ds(wid * IDX_ROWS_PER_WORKER,
                                     IDX_ROWS_PER_WORKER)], idx_v)

    out_base = wid * ROWS_PER_WORKER

    def chunk_body(g, carry):
        buf = lax.rem(g, 2)
        waits = []
        for b in range(GATHERS_PER_CHUNK):
            cp = pltpu.async_copy(
                weight_hbm.at[idx_v.at[g * GATHERS_PER_CHUNK + b]],
                rows_v.at[buf, pl.ds(b * IDX_MINOR, IDX_MINOR)],
                sem_g)
            waits.append(cp)
        for cp in waits:
            cp.wait()
        out_cp = pltpu.async_copy(
            rows_v.at[buf],
            out_hbm.at[pl.ds(out_base + g * CHUNK_ROWS, CHUNK_ROWS)],
            sem_o)
        @pl.when(g > 0)
        def _():
            pltpu.make_async_copy(
                rows_v.at[1 - buf],
                out_hbm.at[pl.ds(out_base + (g - 1) * CHUNK_ROWS, CHUNK_ROWS)],
                sem_o).wait()
        return carry

    lax.fori_loop(0, NUM_CHUNKS, chunk_body, 0)
    last = NUM_CHUNKS - 1
    pltpu.make_async_copy(
        rows_v.at[last % 2],
        out_hbm.at[pl.ds(out_base + last * CHUNK_ROWS, CHUNK_ROWS)],
        sem_o).wait()


@jax.jit
def kernel(x, weight):
    idx2d = x.reshape(ROWS_TOTAL // IDX_MINOR, IDX_MINOR)
    mesh = plsc.VectorSubcoreMesh(core_axis_name="c", subcore_axis_name="s")
    out = pl.kernel(
        _gather_kernel,
        mesh=mesh,
        out_type=jax.ShapeDtypeStruct((ROWS_TOTAL, EMB_DIM), jnp.float32),
        scratch_types=[
            pltpu.VMEM((IDX_ROWS_PER_WORKER, IDX_MINOR), jnp.int32),
            pltpu.VMEM((2, CHUNK_ROWS, EMB_DIM), jnp.float32),
            pltpu.SemaphoreType.DMA,
            pltpu.SemaphoreType.DMA,
        ],
        compiler_params=pltpu.CompilerParams(use_tc_tiling_on_sc=False),
    )(weight, idx2d)
    return out.reshape(BATCH, HIST, EMB_DIM)


# speedup vs baseline: 1.4494x; 1.2204x over previous
import functools

import jax
import jax.numpy as jnp
from jax import lax
from jax.experimental import pallas as pl
from jax.experimental.pallas import tpu as pltpu
from jax.experimental.pallas import tpu_sc as plsc

VOCAB = 1000000
EMB_DIM = 32
BATCH = 4096
HIST = 200

NUM_CORES = 2
NUM_SUBCORES = 16
LANES = 16

BTILE = 128
HBLOCK = 8
NUM_HBLOCKS = HIST // HBLOCK


def _emb_kernel(w_rm, x_t, out5, idx_t, gbuf, pbuf, sem_g0, sem_g1, sem_o):
    c = lax.axis_index("c")
    s = lax.axis_index("s")
    wid = s * NUM_CORES + c
    b0 = wid * BTILE

    iota = lax.iota(jnp.int32, LANES)
    sem_g = (sem_g0, sem_g1)

    def hblock_body(hb, carry):
        pltpu.sync_copy(
            x_t.at[pl.ds(hb * HBLOCK, HBLOCK), pl.ds(b0, BTILE)], idx_t)

        pending = pltpu.async_copy(
            w_rm.at[idx_t.at[0]], gbuf.at[0], sem_g[0])
        for r in range(HBLOCK):
            pending.wait()
            if r + 1 < HBLOCK:
                pending = pltpu.async_copy(
                    w_rm.at[idx_t.at[r + 1]], gbuf.at[(r + 1) % 2],
                    sem_g[(r + 1) % 2])
            g = gbuf.at[r % 2]

            pr = pbuf.at[r]

            def jblk_body(jb, carry2):
                jvec = jb * LANES + iota
                for d in range(EMB_DIM):
                    evec = lax.bitwise_and(iota + d, EMB_DIM - 1)
                    vals = plsc.load_gather(g, [jvec, evec])
                    plsc.store_scatter(
                        pr,
                        [lax.shift_right_logical(evec, 3),
                         lax.bitwise_and(evec, 7), jvec],
                        vals)
                return carry2

            lax.fori_loop(0, BTILE // LANES, jblk_body, 0)
        outs = []
        for r in range(HBLOCK):
            outs.append(pltpu.async_copy(
                pbuf.at[r],
                out5.at[hb * HBLOCK + r, slice(None), wid, slice(None),
                        slice(None)],
                sem_o))
        for cp in outs:
            cp.wait()
        return carry

    lax.fori_loop(0, NUM_HBLOCKS, hblock_body, 0)


@jax.jit
def kernel(x, weight):
    x_t = x.T
    mesh = plsc.VectorSubcoreMesh(core_axis_name="c", subcore_axis_name="s")
    out5 = pl.kernel(
        _emb_kernel,
        mesh=mesh,
        out_type=jax.ShapeDtypeStruct(
            (HIST, EMB_DIM // 8, BATCH // BTILE, 8, BTILE), jnp.float32),
        scratch_types=[
            pltpu.VMEM((HBLOCK, BTILE), jnp.int32),
            pltpu.VMEM((2, BTILE, EMB_DIM), jnp.float32),
            pltpu.VMEM((HBLOCK, EMB_DIM // 8, 8, BTILE),
                       jnp.float32),
            pltpu.SemaphoreType.DMA,
            pltpu.SemaphoreType.DMA,
            pltpu.SemaphoreType.DMA,
        ],
        compiler_params=pltpu.CompilerParams(
            use_tc_tiling_on_sc=False, needs_layout_passes=False),
    )(weight, x_t)
    return jnp.transpose(out5, (2, 4, 0, 1, 3)).reshape(BATCH, HIST, EMB_DIM)


# speedup vs baseline: 1.5162x; 1.0461x over previous
import functools

import jax
import jax.numpy as jnp
from jax import lax
from jax.experimental import pallas as pl
from jax.experimental.pallas import tpu as pltpu
from jax.experimental.pallas import tpu_sc as plsc

VOCAB = 1000000
EMB_DIM = 32
BATCH = 4096
HIST = 200

NUM_CORES = 2
NUM_SUBCORES = 16
LANES = 16

BTILE = 128
HBLOCK = 8
NUM_HBLOCKS = HIST // HBLOCK


def _emb_kernel(w_rm, x_t, out5, idx_t, gbuf, pbuf, sem_g0, sem_g1,
                sem_o0, sem_o1):
    c = lax.axis_index("c")
    s = lax.axis_index("s")
    wid = s * NUM_CORES + c
    b0 = wid * BTILE

    iota = lax.iota(jnp.int32, LANES)
    sem_g = (sem_g0, sem_g1)
    sem_o = (sem_o0, sem_o1)

    def out_copy(hb, par, r):
        return pltpu.make_async_copy(
            pbuf.at[par, r],
            out5.at[hb * HBLOCK + r, slice(None), wid, slice(None),
                    slice(None)],
            sem_o[par])

    def hblock_body(hb, carry):
        pltpu.sync_copy(
            x_t.at[pl.ds(hb * HBLOCK, HBLOCK), pl.ds(b0, BTILE)], idx_t)

        for par in range(2):

            @pl.when(lax.rem(hb, 2) == par)
            def _():
                @pl.when(hb >= 2)
                def _():
                    for r in range(HBLOCK):
                        out_copy(hb - 2, par, r).wait()

                pending = pltpu.async_copy(
                    w_rm.at[idx_t.at[0]], gbuf.at[0], sem_g[0])
                for r in range(HBLOCK):
                    pending.wait()
                    if r + 1 < HBLOCK:
                        pending = pltpu.async_copy(
                            w_rm.at[idx_t.at[r + 1]], gbuf.at[(r + 1) % 2],
                            sem_g[(r + 1) % 2])
                    g = gbuf.at[r % 2]

                    pr = pbuf.at[par, r]

                    def jblk_body(jb, carry2):
                        jvec = jb * LANES + iota
                        for d in range(EMB_DIM):
                            evec = lax.bitwise_and(iota + d, EMB_DIM - 1)
                            vals = plsc.load_gather(g, [jvec, evec])
                            plsc.store_scatter(
                                pr,
                                [lax.shift_right_logical(evec, 3),
                                 lax.bitwise_and(evec, 7), jvec],
                                vals)
                        return carry2

                    lax.fori_loop(0, BTILE // LANES, jblk_body, 0)

                for r in range(HBLOCK):
                    out_copy(hb, par, r).start()

        return carry

    lax.fori_loop(0, NUM_HBLOCKS, hblock_body, 0)
    for hb in (NUM_HBLOCKS - 2, NUM_HBLOCKS - 1):
        for r in range(HBLOCK):
            out_copy(hb, hb % 2, r).wait()


@jax.jit
def kernel(x, weight):
    x_t = x.T
    mesh = plsc.VectorSubcoreMesh(core_axis_name="c", subcore_axis_name="s")
    out5 = pl.kernel(
        _emb_kernel,
        mesh=mesh,
        out_type=jax.ShapeDtypeStruct(
            (HIST, EMB_DIM // 8, BATCH // BTILE, 8, BTILE), jnp.float32),
        scratch_types=[
            pltpu.VMEM((HBLOCK, BTILE), jnp.int32),
            pltpu.VMEM((2, BTILE, EMB_DIM), jnp.float32),
            pltpu.VMEM((2, HBLOCK, EMB_DIM // 8, 8, BTILE),
                       jnp.float32),
            pltpu.SemaphoreType.DMA,
            pltpu.SemaphoreType.DMA,
            pltpu.SemaphoreType.DMA,
            pltpu.SemaphoreType.DMA,
        ],
        compiler_params=pltpu.CompilerParams(
            use_tc_tiling_on_sc=False, needs_layout_passes=False),
    )(weight, x_t)
    return jnp.transpose(out5, (2, 4, 0, 1, 3)).reshape(BATCH, HIST, EMB_DIM)
